# baseline (speedup 1.0000x reference)
import jax
import jax.numpy as jnp
from jax import lax
from jax.experimental import pallas as pl
from jax.experimental.pallas import tpu as pltpu
from jax.experimental.pallas import tpu_sc as plsc

FACTORS = 16
BATCH = 16384
LANES = 16
NUM_ROWS = 1_000_000

_INFO = plsc.get_sparse_core_info()
NUM_CORES = _INFO.num_cores
NUM_SUBCORES = _INFO.num_subcores
NUM_WORKERS = NUM_CORES * NUM_SUBCORES

ROWS_PER_WORKER = BATCH // NUM_WORKERS
CHUNK = 128
NUM_CHUNKS = ROWS_PER_WORKER // CHUNK
GROUPS = ROWS_PER_WORKER // LANES

COL_TILES = NUM_ROWS // CHUNK
EDGE = NUM_ROWS - COL_TILES * CHUNK
NTILES = 2 * (COL_TILES + 1)
SCR_TILES = NTILES + 2
TILES_PER_WORKER = -(-NTILES // NUM_WORKERS)


def _detile_body(ut, it, fu, fi, bounce, sem):
    wid = lax.axis_index("s") * NUM_CORES + lax.axis_index("c")

    def fire(k, carry):
        t = jnp.minimum(wid * TILES_PER_WORKER + k, NTILES - 1)
        band = t // (COL_TILES + 1)
        ct = t % (COL_TILES + 1)
        r0 = pl.multiple_of(jnp.minimum(ct * CHUNK, NUM_ROWS - CHUNK) // CHUNK * CHUNK, CHUNK)
        f0 = pl.multiple_of(band * 8, 8)
        pltpu.async_copy(ut.at[pl.ds(f0, 8), pl.ds(r0, CHUNK)], fu.at[t], sem)
        pltpu.async_copy(it.at[pl.ds(f0, 8), pl.ds(r0, CHUNK)], fi.at[t], sem)
        return carry

    lax.fori_loop(0, TILES_PER_WORKER, fire, 0)

    def drain(k, carry):
        pltpu.make_async_copy(ut.at[pl.ds(0, 8), pl.ds(0, CHUNK)], fu.at[0], sem).wait()
        pltpu.make_async_copy(it.at[pl.ds(0, 8), pl.ds(0, CHUNK)], fi.at[0], sem).wait()
        return carry

    lax.fori_loop(0, TILES_PER_WORKER, drain, 0)

    @pl.when(wid == 0)
    def _():
        for tbl, flat in ((ut, fu), (it, fi)):
            for band in range(2):
                for fo in range(8):
                    pltpu.sync_copy(
                        tbl.at[band * 8 + fo, pl.ds(COL_TILES * CHUNK, EDGE)],
                        bounce.at[fo, pl.ds(0, EDGE)])
                pltpu.sync_copy(bounce, flat.at[NTILES + band])


def _gather_body(fu, fi, w_hbm, b_hbm, uidx_hbm, iidx_hbm, out_hbm,
                 ridx_u, ridx_i, gidx_u, gidx_i, ubuf, ibuf, wtab, bv, outv, sem):
    wid = lax.axis_index("s") * NUM_CORES + lax.axis_index("c")
    base = wid * ROWS_PER_WORKER

    pltpu.sync_copy(uidx_hbm.at[wid], ridx_u)
    pltpu.sync_copy(iidx_hbm.at[wid], ridx_i)
    pltpu.sync_copy(w_hbm, wtab)
    pltpu.sync_copy(b_hbm, bv)

    def idx_body(g, carry):
        off = g * LANES
        for ridx, gidx in ((ridx_u, gidx_u), (ridx_i, gidx_i)):
            r = ridx[pl.ds(off, LANES)]
            rt = r >> 7
            co = r & 127
            is_edge = rt == COL_TILES
            for f in range(FACTORS):
                band = f // 8
                tile = jnp.where(is_edge, NTILES + band, band * (COL_TILES + 1) + rt)
                gidx[f, pl.ds(off, LANES)] = (tile << 10) + (f % 8) * CHUNK + co
        return carry

    lax.fori_loop(0, GROUPS, idx_body, 0)

    copies = []
    for f in range(FACTORS):
        for c in range(NUM_CHUNKS):
            s = pl.ds(c * CHUNK, CHUNK)
            copies.append(pltpu.async_copy(
                fu.at[gidx_u.at[f, s]], ubuf.at[f, s], sem))
            copies.append(pltpu.async_copy(
                fi.at[gidx_i.at[f, s]], ibuf.at[f, s], sem))
    for cp in copies:
        cp.wait()

    bvec = bv[...]
    wspl = [wtab[f, :] for f in range(FACTORS)]

    def acc_body(g, carry):
        off = g * LANES
        acc = bvec
        for f in range(FACTORS):
            acc = acc + ubuf[f, pl.ds(off, LANES)] * ibuf[f, pl.ds(off, LANES)] * wspl[f]
        outv[pl.ds(off, LANES)] = 1.0 / (1.0 + jnp.exp(-acc))
        return carry

    lax.fori_loop(0, GROUPS, acc_body, 0)
    pltpu.sync_copy(outv, out_hbm.at[pl.ds(base, ROWS_PER_WORKER)])


def kernel(user_table, item_table, W, b, user_indices, item_indices):
    ut_t = user_table.T
    it_t = item_table.T
    w_splat = jnp.broadcast_to(W.reshape(FACTORS, 1), (FACTORS, LANES)).astype(jnp.float32)
    b_vec = jnp.broadcast_to(b.reshape(()), (LANES,)).astype(jnp.float32)
    uidx = user_indices.astype(jnp.int32).reshape(NUM_WORKERS, ROWS_PER_WORKER)
    iidx = item_indices.astype(jnp.int32).reshape(NUM_WORKERS, ROWS_PER_WORKER)

    mesh = plsc.VectorSubcoreMesh(core_axis_name="c", subcore_axis_name="s")

    detile = pl.kernel(
        _detile_body,
        mesh=mesh,
        compiler_params=pltpu.CompilerParams(needs_layout_passes=False),
        out_type=(
            jax.ShapeDtypeStruct((SCR_TILES, 8, CHUNK), jnp.float32),
            jax.ShapeDtypeStruct((SCR_TILES, 8, CHUNK), jnp.float32),
        ),
        scratch_types=[
            pltpu.VMEM((8, CHUNK), jnp.float32),
            pltpu.SemaphoreType.DMA,
        ],
    )
    fu3, fi3 = detile(ut_t, it_t)

    gather = pl.kernel(
        _gather_body,
        mesh=mesh,
        compiler_params=pltpu.CompilerParams(
            needs_layout_passes=False, use_tc_tiling_on_sc=False),
        out_type=jax.ShapeDtypeStruct((BATCH,), jnp.float32),
        scratch_types=[
            pltpu.VMEM((ROWS_PER_WORKER,), jnp.int32),
            pltpu.VMEM((ROWS_PER_WORKER,), jnp.int32),
            pltpu.VMEM((FACTORS, ROWS_PER_WORKER), jnp.int32),
            pltpu.VMEM((FACTORS, ROWS_PER_WORKER), jnp.int32),
            pltpu.VMEM((FACTORS, ROWS_PER_WORKER), jnp.float32),
            pltpu.VMEM((FACTORS, ROWS_PER_WORKER), jnp.float32),
            pltpu.VMEM((FACTORS, LANES), jnp.float32),
            pltpu.VMEM((LANES,), jnp.float32),
            pltpu.VMEM((ROWS_PER_WORKER,), jnp.float32),
            pltpu.SemaphoreType.DMA,
        ],
    )
    out = gather(fu3.reshape(-1), fi3.reshape(-1), w_splat, b_vec, uidx, iidx)
    return out.reshape(BATCH, 1)
